# baseline (speedup 1.0000x reference)
import jax
import jax.numpy as jnp
from jax import lax
from jax.experimental import pallas as pl
from jax.experimental.pallas import tpu as pltpu

_N = 1000
_C = 81
_M = 28
_D = _M * _M
_NP = 1024
_BP = 16
_THRESH = 0.5
_F32 = jnp.float32
_HI = jax.lax.Precision.HIGHEST


_BR = _BP * _C


def _gather_body(m_ref, a_ref, x_ref, out_ref):
    prod = x_ref[...] * m_ref[...]
    out_ref[...] = lax.dot_general(
        a_ref[...], prod, (((1,), (0,)), ((), ())),
        precision=_HI, preferred_element_type=_F32)


_gather_call = pl.pallas_call(
    _gather_body,
    grid=(_D // _BP,),
    in_specs=[
        pl.BlockSpec((_BR, _N), lambda i: (0, 0)),
        pl.BlockSpec((_BP, _BR), lambda i: (0, 0)),
        pl.BlockSpec((_BR, _N), lambda i: (i, 0)),
    ],
    out_specs=pl.BlockSpec((_BP, _N), lambda i: (i, 0)),
    out_shape=jax.ShapeDtypeStruct((_D, _N), _F32),
)


def _nms_body(g_ref, srow_ref, scol_ref, out_ref, s_scr):
    masks_t = jax.nn.sigmoid(g_ref[...])
    srow = srow_ref[...]
    scol = scol_ref[...]
    ia = lax.broadcasted_iota(jnp.int32, (_NP, _NP), 0)
    ij = lax.broadcasted_iota(jnp.int32, (_NP, _NP), 1)

    beats = (scol > srow) | ((scol == srow) & (ia < ij))
    rank_row = jnp.sum(beats.astype(_F32), axis=0, keepdims=True)

    rank_i = rank_row.astype(jnp.int32)
    perm = (jnp.broadcast_to(rank_i, (_NP, _NP)) == ia).astype(_F32)
    masks_s = lax.dot_general(
        masks_t, perm, (((1,), (1,)), ((), ())),
        precision=_HI, preferred_element_type=_F32)

    ones_col = jnp.ones((_D, 1), _F32)
    areas = lax.dot_general(
        masks_s, ones_col, (((0,), (0,)), ((), ())),
        precision=_HI, preferred_element_type=_F32)
    inter = lax.dot_general(
        masks_s, masks_s, (((0,), (0,)), ((), ())),
        precision=_HI, preferred_element_type=_F32)
    iou = inter / (areas + 0.0001)
    s_scr[...] = ((iou >= _THRESH) & (ij > ia)).astype(_F32)

    def body(carry):
        keep, _ = carry
        sup = lax.dot_general(
            keep, s_scr[...], (((1,), (0,)), ((), ())),
            preferred_element_type=_F32)
        new = jnp.where(sup > 0.0, 0.0, 1.0)
        changed = jnp.sum(jnp.abs(new - keep)) > 0.0
        return new, changed

    keep0 = jnp.ones((1, _NP), _F32)
    keep, _ = lax.while_loop(lambda c: c[1], body, (keep0, jnp.bool_(True)))

    out_ref[...] = masks_s * keep


_nms_call = pl.pallas_call(
    _nms_body,
    out_shape=jax.ShapeDtypeStruct((_D, _NP), _F32),
    scratch_shapes=[pltpu.VMEM((_NP, _NP), _F32)],
)


def kernel(x, labels, scores):
    x2 = x.transpose(2, 3, 1, 0).reshape(_D * _C, _N)
    oh = (jnp.arange(_C, dtype=jnp.int32)[:, None]
          == labels.astype(jnp.int32)[None, :]).astype(_F32)
    m = jnp.tile(oh, (_BP, 1))
    a = (jnp.arange(_BP, dtype=jnp.int32)[:, None]
         == (jnp.arange(_BR, dtype=jnp.int32)[None, :] // _C)
         ).astype(_F32)
    g = _gather_call(m, a, x2)
    gp = jnp.pad(g, ((0, 0), (0, _NP - _N)))
    sp = jnp.concatenate(
        [scores.astype(_F32), jnp.full((_NP - _N,), -1.0, _F32)])
    out_t = _nms_call(gp, sp.reshape(1, _NP), sp.reshape(_NP, 1))
    return out_t.reshape(_M, _M, _NP).transpose(2, 0, 1)[:_N]


# speedup vs baseline: 2.1189x; 2.1189x over previous
import jax
import jax.numpy as jnp
from jax import lax
from jax.experimental import pallas as pl
from jax.experimental.pallas import tpu as pltpu

_N = 1000
_C = 81
_M = 28
_D = _M * _M
_NP = 1024
_BP = 16
_THRESH = 0.5
_F32 = jnp.float32
_HI = jax.lax.Precision.HIGHEST


def _gather_body(lab_ref, x_ref, out_ref):
    lab = lab_ref[...]
    acc = x_ref[:, 0, :]
    for c in range(1, _C):
        acc = jnp.where(lab == c, x_ref[:, c, :], acc)
    out_ref[...] = acc


_gather_call = pl.pallas_call(
    _gather_body,
    grid=(_D // _BP,),
    in_specs=[
        pl.BlockSpec((1, _N), lambda i: (0, 0)),
        pl.BlockSpec((_BP, _C, _N), lambda i: (i, 0, 0)),
    ],
    out_specs=pl.BlockSpec((_BP, _N), lambda i: (i, 0)),
    out_shape=jax.ShapeDtypeStruct((_D, _N), _F32),
)


def _nms_body(g_ref, srow_ref, scol_ref, out_ref, s_scr):
    masks_t = jax.nn.sigmoid(g_ref[...])
    srow = srow_ref[...]
    scol = scol_ref[...]
    ia = lax.broadcasted_iota(jnp.int32, (_NP, _NP), 0)
    ij = lax.broadcasted_iota(jnp.int32, (_NP, _NP), 1)

    beats = (scol > srow) | ((scol == srow) & (ia < ij))
    rank_row = jnp.sum(beats.astype(_F32), axis=0, keepdims=True)

    rank_i = rank_row.astype(jnp.int32)
    perm = (jnp.broadcast_to(rank_i, (_NP, _NP)) == ia).astype(_F32)
    masks_s = lax.dot_general(
        masks_t, perm, (((1,), (1,)), ((), ())),
        precision=_HI, preferred_element_type=_F32)

    ones_col = jnp.ones((_D, 1), _F32)
    areas = lax.dot_general(
        masks_s, ones_col, (((0,), (0,)), ((), ())),
        precision=_HI, preferred_element_type=_F32)
    inter = lax.dot_general(
        masks_s, masks_s, (((0,), (0,)), ((), ())),
        precision=_HI, preferred_element_type=_F32)
    iou = inter / (areas + 0.0001)
    s_scr[...] = ((iou >= _THRESH) & (ij > ia)).astype(_F32)

    def body(carry):
        keep, _ = carry
        sup = lax.dot_general(
            keep, s_scr[...], (((1,), (0,)), ((), ())),
            preferred_element_type=_F32)
        new = jnp.where(sup > 0.0, 0.0, 1.0)
        changed = jnp.sum(jnp.abs(new - keep)) > 0.0
        return new, changed

    keep0 = jnp.ones((1, _NP), _F32)
    keep, _ = lax.while_loop(lambda c: c[1], body, (keep0, jnp.bool_(True)))

    out_ref[...] = masks_s * keep


_nms_call = pl.pallas_call(
    _nms_body,
    out_shape=jax.ShapeDtypeStruct((_D, _NP), _F32),
    scratch_shapes=[pltpu.VMEM((_NP, _NP), _F32)],
)


def kernel(x, labels, scores):
    x_t = x.transpose(2, 3, 1, 0).reshape(_D, _C, _N)
    g = _gather_call(labels.reshape(1, _N).astype(jnp.int32), x_t)
    gp = jnp.pad(g, ((0, 0), (0, _NP - _N)))
    sp = jnp.concatenate(
        [scores.astype(_F32), jnp.full((_NP - _N,), -1.0, _F32)])
    out_t = _nms_call(gp, sp.reshape(1, _NP), sp.reshape(_NP, 1))
    return out_t.reshape(_M, _M, _NP).transpose(2, 0, 1)[:_N]
